# baseline (speedup 1.0000x reference)
import jax
import jax.numpy as jnp
from jax.experimental import pallas as pl
from jax.experimental.pallas import tpu as pltpu


def _copy_kernel(x_ref, o_ref):
    o_ref[...] = x_ref[...]


def kernel(x, w1_t, w2_t):
    B, C, H, W = x.shape
    HW = H * W
    R = 64
    L = C * HW // R
    BB = 4
    xr = x.reshape(B, R, L)
    out = pl.pallas_call(
        _copy_kernel,
        out_shape=jax.ShapeDtypeStruct((B, R, L), x.dtype),
        grid=(B // BB,),
        in_specs=[pl.BlockSpec((BB, R, L), lambda b: (b, 0, 0))],
        out_specs=pl.BlockSpec((BB, R, L), lambda b: (b, 0, 0)),
        compiler_params=pltpu.CompilerParams(
            dimension_semantics=("parallel",),
        ),
    )(xr)
    return out.reshape(B, C, H, W)


# speedup vs baseline: 3.2885x; 3.2885x over previous
import functools

import jax
import jax.numpy as jnp
from jax import lax
from jax.experimental import pallas as pl
from jax.experimental.pallas import tpu as pltpu


def _se_kernel(x_ref, w1_ref, w2_ref, o_ref, *, inv_hw):
    xf = x_ref[...]
    pooled = jnp.sum(xf, axis=-1, keepdims=True) * inv_hw
    for i in range(xf.shape[0]):
        y = pooled[i]
        h = lax.dot_general(w1_ref[...], y, (((1,), (0,)), ((), ())),
                            preferred_element_type=jnp.float32)
        h = jnp.maximum(h, 0.0)
        logits = lax.dot_general(w2_ref[...], h, (((1,), (0,)), ((), ())),
                                 preferred_element_type=jnp.float32)
        gates = jax.nn.sigmoid(logits)
        o_ref[i] = (xf[i] * gates).astype(o_ref.dtype)


def kernel(x, w1_t, w2_t):
    B, C, H, W = x.shape
    Cr = w1_t.shape[1]
    HW = H * W
    xr = x.reshape(B, C, HW)
    w1 = w1_t.T.astype(jnp.float32)
    w2 = w2_t.T.astype(jnp.float32)

    slab_bytes = C * ((HW + 127) // 128 * 128) * x.dtype.itemsize
    BB = 1
    for cand in (8, 4, 2):
        if B % cand == 0 and 4 * cand * slab_bytes <= 61 << 20:
            BB = cand
            break

    out = pl.pallas_call(
        functools.partial(_se_kernel, inv_hw=1.0 / float(HW)),
        out_shape=jax.ShapeDtypeStruct((B, C, HW), x.dtype),
        grid=(B // BB,),
        in_specs=[
            pl.BlockSpec((BB, C, HW), lambda b: (b, 0, 0)),
            pl.BlockSpec((Cr, C), lambda b: (0, 0)),
            pl.BlockSpec((C, Cr), lambda b: (0, 0)),
        ],
        out_specs=pl.BlockSpec((BB, C, HW), lambda b: (b, 0, 0)),
        compiler_params=pltpu.CompilerParams(
            dimension_semantics=("parallel",),
            vmem_limit_bytes=63 << 20,
        ),
        cost_estimate=pl.CostEstimate(
            flops=2 * B * C * HW + 4 * B * C * Cr,
            transcendentals=B * C,
            bytes_accessed=2 * B * C * HW * x.dtype.itemsize,
        ),
    )(xr, w1, w2)
    return out.reshape(B, C, H, W)


# speedup vs baseline: 3.2915x; 1.0009x over previous
import functools

import jax
import jax.numpy as jnp
from jax import lax
from jax.experimental import pallas as pl
from jax.experimental.pallas import tpu as pltpu


def _se_kernel(x_ref, w1_ref, w2_ref, o_ref, *, inv_hw):
    xf = x_ref[...]
    pooled = jnp.sum(xf, axis=-1, keepdims=True) * inv_hw
    for i in range(xf.shape[0]):
        y = pooled[i]
        h = lax.dot_general(w1_ref[...], y, (((1,), (0,)), ((), ())),
                            preferred_element_type=jnp.float32)
        h = jnp.maximum(h, 0.0)
        logits = lax.dot_general(w2_ref[...], h, (((1,), (0,)), ((), ())),
                                 preferred_element_type=jnp.float32)
        gates = jax.nn.sigmoid(logits)
        o_ref[i] = (xf[i] * gates).astype(o_ref.dtype)


def kernel(x, w1_t, w2_t):
    B, C, H, W = x.shape
    Cr = w1_t.shape[1]
    HW = H * W
    xr = x.reshape(B, C, HW)
    w1 = w1_t.T.astype(jnp.float32)
    w2 = w2_t.T.astype(jnp.float32)

    slab_bytes = C * ((HW + 127) // 128 * 128) * x.dtype.itemsize
    BB = 1
    for cand in (8, 4, 2):
        if B % cand == 0 and 4 * cand * slab_bytes <= 61 << 20:
            BB = cand
            break

    out = pl.pallas_call(
        functools.partial(_se_kernel, inv_hw=1.0 / float(HW)),
        out_shape=jax.ShapeDtypeStruct((B, C, HW), x.dtype),
        grid=(B // BB,),
        in_specs=[
            pl.BlockSpec((BB, C, HW), lambda b: (b, 0, 0)),
            pl.BlockSpec((Cr, C), lambda b: (0, 0)),
            pl.BlockSpec((C, Cr), lambda b: (0, 0)),
        ],
        out_specs=pl.BlockSpec((BB, C, HW), lambda b: (b, 0, 0)),
        compiler_params=pltpu.CompilerParams(
            dimension_semantics=("arbitrary",),
            vmem_limit_bytes=63 << 20,
        ),
        cost_estimate=pl.CostEstimate(
            flops=2 * B * C * HW + 4 * B * C * Cr,
            transcendentals=B * C,
            bytes_accessed=2 * B * C * HW * x.dtype.itemsize,
        ),
    )(xr, w1, w2)
    return out.reshape(B, C, H, W)
